# baseline (speedup 1.0000x reference)
import functools

import jax
import jax.numpy as jnp
from jax import lax
from jax.experimental import pallas as pl
from jax.experimental.pallas import tpu as pltpu
from jax.experimental.pallas import tpu_sc as plsc

NC = 2
NS = 16
NW = NC * NS

C = 8000
STAGE = 10000


def _pairwise_sc(rx, ry, rz, ox, oy, oz, idx_i, idx_j):
    E = idx_i.shape[0]
    n = rx.shape[0]
    per_w = E // NW
    n_chunks = per_w // C
    mesh = plsc.VectorSubcoreMesh(core_axis_name="c", subcore_axis_name="s")

    @functools.partial(
        pl.kernel,
        mesh=mesh,
        compiler_params=pltpu.CompilerParams(use_tc_tiling_on_sc=False),
        out_type=tuple(
            jax.ShapeDtypeStruct((E,), jnp.float32) for _ in range(3)
        ),
        scratch_types=[
            pltpu.VMEM((C,), jnp.int32),
            pltpu.VMEM((C,), jnp.int32),
            pltpu.VMEM((C,), jnp.float32),
            pltpu.VMEM((C,), jnp.float32),
            pltpu.VMEM((C,), jnp.float32),
            pltpu.VMEM((C,), jnp.float32),
            pltpu.VMEM((C,), jnp.float32),
            pltpu.VMEM((C,), jnp.float32),
            pltpu.VMEM((C,), jnp.float32),
            pltpu.VMEM((C,), jnp.float32),
            pltpu.VMEM((C,), jnp.float32),
            pltpu.VMEM_SHARED((n,), jnp.float32),
            pltpu.VMEM_SHARED((n,), jnp.float32),
            pltpu.VMEM_SHARED((n,), jnp.float32),
            pltpu.SemaphoreType.DMA,
        ],
    )
    def k(rx_h, ry_h, rz_h, ox_h, oy_h, oz_h, ii_h, jj_h,
          cx_h, cy_h, cz_h,
          ii_v, jj_v, gix, giy, giz, gjx, gjy, gjz, ofx, ofy, ofz,
          rx_sp, ry_sp, rz_sp, sem):
        sid = lax.axis_index("s")
        wid = sid * NC + lax.axis_index("c")

        @pl.when(sid < n // STAGE)
        def _():
            sbase = sid * STAGE
            pltpu.sync_copy(rx_h.at[pl.ds(sbase, STAGE)],
                            rx_sp.at[pl.ds(sbase, STAGE)])
            pltpu.sync_copy(ry_h.at[pl.ds(sbase, STAGE)],
                            ry_sp.at[pl.ds(sbase, STAGE)])
            pltpu.sync_copy(rz_h.at[pl.ds(sbase, STAGE)],
                            rz_sp.at[pl.ds(sbase, STAGE)])

        plsc.subcore_barrier()

        def chunk_body(t, carry):
            base = wid * per_w + t * C
            pltpu.sync_copy(ii_h.at[pl.ds(base, C)], ii_v)
            pltpu.sync_copy(jj_h.at[pl.ds(base, C)], jj_v)
            cps = [
                pltpu.async_copy(rx_sp.at[ii_v], gix, sem),
                pltpu.async_copy(ry_sp.at[ii_v], giy, sem),
                pltpu.async_copy(rz_sp.at[ii_v], giz, sem),
                pltpu.async_copy(rx_sp.at[jj_v], gjx, sem),
                pltpu.async_copy(ry_sp.at[jj_v], gjy, sem),
                pltpu.async_copy(rz_sp.at[jj_v], gjz, sem),
            ]
            pltpu.sync_copy(ox_h.at[pl.ds(base, C)], ofx)
            pltpu.sync_copy(oy_h.at[pl.ds(base, C)], ofy)
            pltpu.sync_copy(oz_h.at[pl.ds(base, C)], ofz)
            for cp in cps:
                cp.wait()

            def vec_body(m, carry2):
                s = pl.ds(m * 16, 16)
                gix[s] = gjx[s] - gix[s] + ofx[s]
                giy[s] = gjy[s] - giy[s] + ofy[s]
                giz[s] = gjz[s] - giz[s] + ofz[s]
                return carry2

            lax.fori_loop(0, C // 16, vec_body, 0, unroll=4)
            pltpu.sync_copy(gix, cx_h.at[pl.ds(base, C)])
            pltpu.sync_copy(giy, cy_h.at[pl.ds(base, C)])
            pltpu.sync_copy(giz, cz_h.at[pl.ds(base, C)])
            return carry

        lax.fori_loop(0, n_chunks, chunk_body, 0)

    return k(rx, ry, rz, ox, oy, oz, idx_i, idx_j)


def kernel(R, offsets, idx_i, idx_j):
    E = idx_i.shape[0]
    cx, cy, cz = _pairwise_sc(
        R[:, 0], R[:, 1], R[:, 2],
        offsets[:, 0], offsets[:, 1], offsets[:, 2],
        idx_i.astype(jnp.int32), idx_j.astype(jnp.int32))
    return jnp.stack([cx, cy, cz], axis=1)


# speedup vs baseline: 1.2830x; 1.2830x over previous
import functools

import jax
import jax.numpy as jnp
from jax import lax
from jax.experimental import pallas as pl
from jax.experimental.pallas import tpu as pltpu
from jax.experimental.pallas import tpu_sc as plsc

NC = 2
NS = 16
NW = NC * NS

C = 4000
STAGE = 10000


def _pairwise_sc(rx, ry, rz, ox, oy, oz, idx_i, idx_j):
    E = idx_i.shape[0]
    n = rx.shape[0]
    per_w = E // NW
    n_chunks = per_w // C
    half = n_chunks // 2
    mesh = plsc.VectorSubcoreMesh(core_axis_name="c", subcore_axis_name="s")

    buf = lambda: pltpu.VMEM((C,), jnp.float32)
    ibuf = lambda: pltpu.VMEM((C,), jnp.int32)

    @functools.partial(
        pl.kernel,
        mesh=mesh,
        compiler_params=pltpu.CompilerParams(use_tc_tiling_on_sc=False),
        out_type=tuple(
            jax.ShapeDtypeStruct((E,), jnp.float32) for _ in range(3)
        ),
        scratch_types=(
            [ibuf(), ibuf()] + [buf() for _ in range(9)]
            + [ibuf(), ibuf()] + [buf() for _ in range(9)]
            + [
                pltpu.VMEM_SHARED((n,), jnp.float32),
                pltpu.VMEM_SHARED((n,), jnp.float32),
                pltpu.VMEM_SHARED((n,), jnp.float32),
                pltpu.SemaphoreType.DMA,
                pltpu.SemaphoreType.DMA,
            ]
        ),
    )
    def k(rx_h, ry_h, rz_h, ox_h, oy_h, oz_h, ii_h, jj_h,
          cx_h, cy_h, cz_h, *bufs):
        a = bufs[0:11]
        b = bufs[11:22]
        rx_sp, ry_sp, rz_sp, sem_a, sem_b = bufs[22:27]
        sid = lax.axis_index("s")
        wid = sid * NC + lax.axis_index("c")

        @pl.when(sid < n // STAGE)
        def _():
            sbase = sid * STAGE
            pltpu.sync_copy(rx_h.at[pl.ds(sbase, STAGE)],
                            rx_sp.at[pl.ds(sbase, STAGE)])
            pltpu.sync_copy(ry_h.at[pl.ds(sbase, STAGE)],
                            ry_sp.at[pl.ds(sbase, STAGE)])
            pltpu.sync_copy(rz_h.at[pl.ds(sbase, STAGE)],
                            rz_sp.at[pl.ds(sbase, STAGE)])

        plsc.subcore_barrier()

        def fire(t, s, sem):
            ii_v, jj_v = s[0], s[1]
            base = wid * per_w + t * C
            pltpu.sync_copy(ii_h.at[pl.ds(base, C)], ii_v)
            pltpu.sync_copy(jj_h.at[pl.ds(base, C)], jj_v)
            cps = [
                pltpu.async_copy(rx_sp.at[ii_v], s[2], sem),
                pltpu.async_copy(ry_sp.at[ii_v], s[3], sem),
                pltpu.async_copy(rz_sp.at[ii_v], s[4], sem),
                pltpu.async_copy(rx_sp.at[jj_v], s[5], sem),
                pltpu.async_copy(ry_sp.at[jj_v], s[6], sem),
                pltpu.async_copy(rz_sp.at[jj_v], s[7], sem),
            ]
            pltpu.sync_copy(ox_h.at[pl.ds(base, C)], s[8])
            pltpu.sync_copy(oy_h.at[pl.ds(base, C)], s[9])
            pltpu.sync_copy(oz_h.at[pl.ds(base, C)], s[10])
            return cps

        def finish(t, s, cps):
            gix, giy, giz, gjx, gjy, gjz, ofx, ofy, ofz = s[2:11]
            for cp in cps:
                cp.wait()

            def vec_body(m, carry2):
                sl = pl.ds(m * 16, 16)
                gix[sl] = gjx[sl] - gix[sl] + ofx[sl]
                giy[sl] = gjy[sl] - giy[sl] + ofy[sl]
                giz[sl] = gjz[sl] - giz[sl] + ofz[sl]
                return carry2

            lax.fori_loop(0, C // 16, vec_body, 0, unroll=4)
            base = wid * per_w + t * C
            pltpu.sync_copy(gix, cx_h.at[pl.ds(base, C)])
            pltpu.sync_copy(giy, cy_h.at[pl.ds(base, C)])
            pltpu.sync_copy(giz, cz_h.at[pl.ds(base, C)])

        cps0 = fire(0, a, sem_a)

        def body(u, carry):
            t_a = u * 2
            cps_b = fire(t_a + 1, b, sem_b)
            finish(t_a, a, cps0)

            @pl.when(u + 1 < half)
            def _():
                fire(t_a + 2, a, sem_a)

            finish(t_a + 1, b, cps_b)
            return carry

        lax.fori_loop(0, half, body, 0)

    return k(rx, ry, rz, ox, oy, oz, idx_i, idx_j)


def kernel(R, offsets, idx_i, idx_j):
    cx, cy, cz = _pairwise_sc(
        R[:, 0], R[:, 1], R[:, 2],
        offsets[:, 0], offsets[:, 1], offsets[:, 2],
        idx_i.astype(jnp.int32), idx_j.astype(jnp.int32))
    return jnp.stack([cx, cy, cz], axis=1)


# speedup vs baseline: 1.5752x; 1.2278x over previous
import functools

import jax
import jax.numpy as jnp
from jax import lax
from jax.experimental import pallas as pl
from jax.experimental.pallas import tpu as pltpu
from jax.experimental.pallas import tpu_sc as plsc

NC = 2
NS = 16
NW = NC * NS

C = 4000
STAGE = 10000


def _gather_sc(rx, ry, rz, idx_i, idx_j):
    E = idx_i.shape[0]
    n = rx.shape[0]
    per_w = E // NW
    n_chunks = per_w // C
    half = n_chunks // 2
    mesh = plsc.VectorSubcoreMesh(core_axis_name="c", subcore_axis_name="s")

    buf = lambda: pltpu.VMEM((C,), jnp.float32)
    ibuf = lambda: pltpu.VMEM((C,), jnp.int32)

    @functools.partial(
        pl.kernel,
        mesh=mesh,
        compiler_params=pltpu.CompilerParams(use_tc_tiling_on_sc=False),
        out_type=tuple(
            jax.ShapeDtypeStruct((E,), jnp.float32) for _ in range(6)
        ),
        scratch_types=(
            [ibuf(), ibuf()] + [buf() for _ in range(6)]
            + [ibuf(), ibuf()] + [buf() for _ in range(6)]
            + [
                pltpu.VMEM_SHARED((n,), jnp.float32),
                pltpu.VMEM_SHARED((n,), jnp.float32),
                pltpu.VMEM_SHARED((n,), jnp.float32),
                pltpu.SemaphoreType.DMA,
                pltpu.SemaphoreType.DMA,
            ]
        ),
    )
    def k(rx_h, ry_h, rz_h, ii_h, jj_h,
          ix_h, iy_h, iz_h, jx_h, jy_h, jz_h, *bufs):
        a = bufs[0:8]
        b = bufs[8:16]
        rx_sp, ry_sp, rz_sp, sem_a, sem_b = bufs[16:21]
        sid = lax.axis_index("s")
        wid = sid * NC + lax.axis_index("c")

        @pl.when(sid < n // STAGE)
        def _():
            sbase = sid * STAGE
            pltpu.sync_copy(rx_h.at[pl.ds(sbase, STAGE)],
                            rx_sp.at[pl.ds(sbase, STAGE)])
            pltpu.sync_copy(ry_h.at[pl.ds(sbase, STAGE)],
                            ry_sp.at[pl.ds(sbase, STAGE)])
            pltpu.sync_copy(rz_h.at[pl.ds(sbase, STAGE)],
                            rz_sp.at[pl.ds(sbase, STAGE)])

        plsc.subcore_barrier()

        def fire(t, s, sem):
            ii_v, jj_v = s[0], s[1]
            base = wid * per_w + t * C
            pltpu.sync_copy(ii_h.at[pl.ds(base, C)], ii_v)
            pltpu.sync_copy(jj_h.at[pl.ds(base, C)], jj_v)
            return [
                pltpu.async_copy(rx_sp.at[ii_v], s[2], sem),
                pltpu.async_copy(ry_sp.at[ii_v], s[3], sem),
                pltpu.async_copy(rz_sp.at[ii_v], s[4], sem),
                pltpu.async_copy(rx_sp.at[jj_v], s[5], sem),
                pltpu.async_copy(ry_sp.at[jj_v], s[6], sem),
                pltpu.async_copy(rz_sp.at[jj_v], s[7], sem),
            ]

        def finish(t, s, cps):
            for cp in cps:
                cp.wait()
            base = wid * per_w + t * C
            pltpu.sync_copy(s[2], ix_h.at[pl.ds(base, C)])
            pltpu.sync_copy(s[3], iy_h.at[pl.ds(base, C)])
            pltpu.sync_copy(s[4], iz_h.at[pl.ds(base, C)])
            pltpu.sync_copy(s[5], jx_h.at[pl.ds(base, C)])
            pltpu.sync_copy(s[6], jy_h.at[pl.ds(base, C)])
            pltpu.sync_copy(s[7], jz_h.at[pl.ds(base, C)])

        cps0 = fire(0, a, sem_a)

        def body(u, carry):
            t_a = u * 2
            cps_b = fire(t_a + 1, b, sem_b)
            finish(t_a, a, cps0)

            @pl.when(u + 1 < half)
            def _():
                fire(t_a + 2, a, sem_a)

            finish(t_a + 1, b, cps_b)
            return carry

        lax.fori_loop(0, half, body, 0)

    return k(rx, ry, rz, idx_i, idx_j)


def _combine_body(ax_r, ay_r, az_r, bx_r, by_r, bz_r, ox_r, oy_r, oz_r,
                  cx_r, cy_r, cz_r):
    cx_r[...] = bx_r[...] - ax_r[...] + ox_r[...]
    cy_r[...] = by_r[...] - ay_r[...] + oy_r[...]
    cz_r[...] = bz_r[...] - az_r[...] + oz_r[...]


def _combine_tc(pi, pj, off):
    rows = pi[0].shape[0]
    br = 2000
    grid = rows // br
    spec = pl.BlockSpec((br, 128), lambda i: (i, 0))
    one = jax.ShapeDtypeStruct((rows, 128), jnp.float32)
    return pl.pallas_call(
        _combine_body,
        out_shape=(one, one, one),
        grid=(grid,),
        in_specs=[spec] * 9,
        out_specs=(spec, spec, spec),
    )(*pi, *pj, *off)


def kernel(R, offsets, idx_i, idx_j):
    E = idx_i.shape[0]
    rows = E // 128
    ix, iy, iz, jx, jy, jz = _gather_sc(
        R[:, 0], R[:, 1], R[:, 2],
        idx_i.astype(jnp.int32), idx_j.astype(jnp.int32))
    pi = [a.reshape(rows, 128) for a in (ix, iy, iz)]
    pj = [a.reshape(rows, 128) for a in (jx, jy, jz)]
    off = [offsets[:, c].reshape(rows, 128) for c in range(3)]
    cx, cy, cz = _combine_tc(pi, pj, off)
    return jnp.stack([cx.reshape(E), cy.reshape(E), cz.reshape(E)], axis=1)
